# Initial kernel scaffold; baseline (speedup 1.0000x reference)
import functools

import jax
import jax.numpy as jnp
from jax import lax
from jax.experimental import pallas as pl
from jax.experimental.pallas import tpu as pltpu
from jax.experimental.pallas import tpu_sc as plsc

B = 16384
EMB = 32
COLOR_LEN = 16
TITLE_LEN = 32

_INFO = plsc.get_sparse_core_info()
NC = _INFO.num_cores
NS = _INFO.num_subcores
NW = NC * NS
ROWS_PER_W = B // NW
CHUNK = 64
NCHUNK = ROWS_PER_W // CHUNK
IDXW = 128


def _sc_body(id_idx_hbm, color_idx_hbm, title_idx_hbm,
             id_tab_hbm, color_tab_hbm, title_tab_hbm, out_hbm,
             id_idx_v, color_idx_v, title_idx_v,
             id_rows_v, color_rows_v, title_rows_v, out_v, sem):
    wid = lax.axis_index("s") * NC + lax.axis_index("c")
    wbase = wid * ROWS_PER_W

    for g in range(NCHUNK):
        base = wbase + g * CHUNK
        pltpu.sync_copy(id_idx_hbm.at[pl.ds(base, CHUNK)], id_idx_v)
        pltpu.sync_copy(
            color_idx_hbm.at[pl.ds(base * COLOR_LEN // IDXW,
                                   CHUNK * COLOR_LEN // IDXW)],
            color_idx_v)
        pltpu.sync_copy(
            title_idx_hbm.at[pl.ds(base * TITLE_LEN // IDXW,
                                   CHUNK * TITLE_LEN // IDXW)],
            title_idx_v)

        copies = [pltpu.async_copy(id_tab_hbm.at[id_idx_v], id_rows_v, sem)]
        for j in range(CHUNK * COLOR_LEN // IDXW):
            copies.append(pltpu.async_copy(
                color_tab_hbm.at[color_idx_v.at[j]],
                color_rows_v.at[pl.ds(j * IDXW, IDXW)], sem))
        for j in range(CHUNK * TITLE_LEN // IDXW):
            copies.append(pltpu.async_copy(
                title_tab_hbm.at[title_idx_v.at[j]],
                title_rows_v.at[pl.ds(j * IDXW, IDXW)], sem))
        for cp in copies:
            cp.wait()

        def sample_body(i, carry):
            rc = i * COLOR_LEN
            rt = i * TITLE_LEN
            for j in range(EMB // 16):
                sl = pl.ds(j * 16, 16)
                out_v[i, pl.ds(j * 16, 16)] = id_rows_v[i, sl]
                p = [color_rows_v[rc + t, sl] for t in range(4)]
                for t in range(4, COLOR_LEN):
                    p[t % 4] = p[t % 4] + color_rows_v[rc + t, sl]
                csum = (p[0] + p[1]) + (p[2] + p[3])
                out_v[i, pl.ds(EMB + j * 16, 16)] = csum * (1.0 / COLOR_LEN)
                q = [title_rows_v[rt + t, sl] for t in range(4)]
                for t in range(4, TITLE_LEN):
                    q[t % 4] = q[t % 4] + title_rows_v[rt + t, sl]
                tsum = (q[0] + q[1]) + (q[2] + q[3])
                out_v[i, pl.ds(2 * EMB + j * 16, 16)] = tsum * (1.0 / TITLE_LEN)
            return carry

        lax.fori_loop(0, CHUNK, sample_body, 0)
        pltpu.sync_copy(out_v, out_hbm.at[pl.ds(base, CHUNK)])


@jax.jit
def _sc_call(item_id, color_idx, title_idx, id_table, color_table, title_table):
    f = functools.partial(
        pl.kernel,
        out_type=jax.ShapeDtypeStruct((B, 3 * EMB), jnp.float32),
        mesh=plsc.VectorSubcoreMesh(core_axis_name="c", subcore_axis_name="s"),
        scratch_types=[
            pltpu.VMEM((CHUNK,), jnp.int32),
            pltpu.VMEM((CHUNK * COLOR_LEN // IDXW, IDXW), jnp.int32),
            pltpu.VMEM((CHUNK * TITLE_LEN // IDXW, IDXW), jnp.int32),
            pltpu.VMEM((CHUNK, EMB), jnp.float32),
            pltpu.VMEM((CHUNK * COLOR_LEN, EMB), jnp.float32),
            pltpu.VMEM((CHUNK * TITLE_LEN, EMB), jnp.float32),
            pltpu.VMEM((CHUNK, 3 * EMB), jnp.float32),
            pltpu.SemaphoreType.DMA,
        ],
    )(_sc_body)
    return f(item_id, color_idx, title_idx, id_table, color_table, title_table)


def kernel(item_id, color_tokens, title_tokens, id_table, color_table, title_table):
    color_idx = color_tokens.reshape(B * COLOR_LEN // IDXW, IDXW)
    title_idx = title_tokens.reshape(B * TITLE_LEN // IDXW, IDXW)
    return _sc_call(item_id, color_idx, title_idx,
                    id_table, color_table, title_table)


# speedup vs baseline: 5.0231x; 5.0231x over previous
import functools

import jax
import jax.numpy as jnp
from jax import lax
from jax.experimental import pallas as pl
from jax.experimental.pallas import tpu as pltpu
from jax.experimental.pallas import tpu_sc as plsc

B = 16384
EMB = 32
COLOR_LEN = 16
TITLE_LEN = 32

NC = 2
NS = 16
NW = NC * NS
ROWS_PER_W = B // NW
CHUNK = 64
NCHUNK = ROWS_PER_W // CHUNK
IDXW = 128


def _sc_body(id_idx_hbm, color_idx_hbm, title_idx_hbm,
             id_tab_hbm, color_tab_hbm, title_tab_hbm, out_hbm,
             id_idx_v, color_idx_v, title_idx_v,
             id_rows_v, color_rows_v, title_rows_v, out_v, sem):
    wid = lax.axis_index("s") * NC + lax.axis_index("c")
    wbase = wid * ROWS_PER_W

    for g in range(NCHUNK):
        base = pl.multiple_of(wbase + g * CHUNK, CHUNK)
        cbase = pl.multiple_of(base * COLOR_LEN // IDXW, CHUNK * COLOR_LEN // IDXW)
        tbase = pl.multiple_of(base * TITLE_LEN // IDXW, CHUNK * TITLE_LEN // IDXW)
        pltpu.sync_copy(id_idx_hbm.at[pl.ds(base, CHUNK)], id_idx_v)
        pltpu.sync_copy(
            color_idx_hbm.at[pl.ds(cbase, CHUNK * COLOR_LEN // IDXW)],
            color_idx_v)
        pltpu.sync_copy(
            title_idx_hbm.at[pl.ds(tbase, CHUNK * TITLE_LEN // IDXW)],
            title_idx_v)

        copies = [pltpu.async_copy(id_tab_hbm.at[id_idx_v], id_rows_v, sem)]
        for j in range(CHUNK * COLOR_LEN // IDXW):
            copies.append(pltpu.async_copy(
                color_tab_hbm.at[color_idx_v.at[j]],
                color_rows_v.at[pl.ds(j * IDXW, IDXW)], sem))
        for j in range(CHUNK * TITLE_LEN // IDXW):
            copies.append(pltpu.async_copy(
                title_tab_hbm.at[title_idx_v.at[j]],
                title_rows_v.at[pl.ds(j * IDXW, IDXW)], sem))
        for cp in copies:
            cp.wait()

        def sample_body(i, carry):
            rc = i * COLOR_LEN
            rt = i * TITLE_LEN
            for j in range(EMB // 16):
                sl = pl.ds(j * 16, 16)
                out_v[i, pl.ds(j * 16, 16)] = id_rows_v[i, sl]
                p = [color_rows_v[rc + t, sl] for t in range(4)]
                for t in range(4, COLOR_LEN):
                    p[t % 4] = p[t % 4] + color_rows_v[rc + t, sl]
                csum = (p[0] + p[1]) + (p[2] + p[3])
                out_v[i, pl.ds(EMB + j * 16, 16)] = csum * (1.0 / COLOR_LEN)
                q = [title_rows_v[rt + t, sl] for t in range(4)]
                for t in range(4, TITLE_LEN):
                    q[t % 4] = q[t % 4] + title_rows_v[rt + t, sl]
                tsum = (q[0] + q[1]) + (q[2] + q[3])
                out_v[i, pl.ds(2 * EMB + j * 16, 16)] = tsum * (1.0 / TITLE_LEN)
            return carry

        lax.fori_loop(0, CHUNK, sample_body, 0)
        pltpu.sync_copy(out_v, out_hbm.at[pl.ds(base, CHUNK)])


@jax.jit
def _sc_call(item_id, color_idx, title_idx, id_table, color_table, title_table):
    f = functools.partial(
        pl.kernel,
        out_type=jax.ShapeDtypeStruct((B, 3 * EMB), jnp.float32),
        mesh=plsc.VectorSubcoreMesh(core_axis_name="c", subcore_axis_name="s"),
        scratch_types=[
            pltpu.VMEM((CHUNK,), jnp.int32),
            pltpu.VMEM((CHUNK * COLOR_LEN // IDXW, IDXW), jnp.int32),
            pltpu.VMEM((CHUNK * TITLE_LEN // IDXW, IDXW), jnp.int32),
            pltpu.VMEM((CHUNK, EMB), jnp.float32),
            pltpu.VMEM((CHUNK * COLOR_LEN, EMB), jnp.float32),
            pltpu.VMEM((CHUNK * TITLE_LEN, EMB), jnp.float32),
            pltpu.VMEM((CHUNK, 3 * EMB), jnp.float32),
            pltpu.SemaphoreType.DMA,
        ],
        compiler_params=pltpu.CompilerParams(use_tc_tiling_on_sc=False),
    )(_sc_body)
    return f(item_id, color_idx, title_idx, id_table, color_table, title_table)


def kernel(item_id, color_tokens, title_tokens, id_table, color_table, title_table):
    color_idx = color_tokens.reshape(B * COLOR_LEN // IDXW, IDXW)
    title_idx = title_tokens.reshape(B * TITLE_LEN // IDXW, IDXW)
    return _sc_call(item_id, color_idx, title_idx,
                    id_table, color_table, title_table)
